# baseline (speedup 1.0000x reference)
import numpy as np
import jax
import jax.numpy as jnp
from jax import lax
from jax.experimental import pallas as pl
from jax.experimental.pallas import tpu as pltpu
from jax.experimental.pallas import tpu_sc as plsc

NNODE = 10000
NEDGE = 160000
MULQ = 64
DNODE = 256
DMSG = 512
ACTN = 1.679177

NCORE = 2
NSUB = 16
NWORK = NCORE * NSUB

CHUNKS = (80128, 79872)


GCH = 128


def _make_gather(ec):
    assert ec % (NWORK * 8) == 0
    ew = ec // NWORK
    nfull, tail = divmod(ew, GCH)
    npair, odd = divmod(nfull, 2)

    def body(nodes, srcidx, xe, idx_v, rows0, rows1, gs0, gs1, ws0, ws1):
        cid = lax.axis_index("c")
        sid = lax.axis_index("s")
        wid = sid * NCORE + cid
        base = pl.multiple_of(wid * ew, 8)
        pltpu.sync_copy(srcidx.at[pl.ds(base, ew)], idx_v)

        def loop(i, carry):
            off = pl.multiple_of(i * (2 * GCH), 2 * GCH)

            @pl.when(i > 0)
            def _():
                pltpu.make_async_copy(rows0, xe.at[wid, pl.ds(0, GCH)], ws0).wait()

            g0 = pltpu.async_copy(nodes.at[idx_v.at[pl.ds(off, GCH)]], rows0, gs0)

            @pl.when(i > 0)
            def _():
                pltpu.make_async_copy(rows1, xe.at[wid, pl.ds(0, GCH)], ws1).wait()

            g1 = pltpu.async_copy(nodes.at[idx_v.at[pl.ds(off + GCH, GCH)]], rows1, gs1)
            g0.wait()
            pltpu.async_copy(rows0, xe.at[wid, pl.ds(off, GCH)], ws0)
            g1.wait()
            pltpu.async_copy(rows1, xe.at[wid, pl.ds(off + GCH, GCH)], ws1)
            return carry

        if npair:
            lax.fori_loop(0, npair, loop, 0)
            pltpu.make_async_copy(rows0, xe.at[wid, pl.ds(0, GCH)], ws0).wait()
            pltpu.make_async_copy(rows1, xe.at[wid, pl.ds(0, GCH)], ws1).wait()
        off = npair * 2 * GCH
        if odd:
            g = pltpu.async_copy(nodes.at[idx_v.at[pl.ds(off, GCH)]], rows0, gs0)
            g.wait()
            pltpu.sync_copy(rows0, xe.at[wid, pl.ds(off, GCH)])
            off += GCH
        if tail:
            g = pltpu.async_copy(nodes.at[idx_v.at[pl.ds(off, tail)]],
                                 rows1.at[pl.ds(0, tail)], gs1)
            g.wait()
            pltpu.sync_copy(rows1.at[pl.ds(0, tail)], xe.at[wid, pl.ds(off, tail)])

    return pl.kernel(
        body,
        out_type=jax.ShapeDtypeStruct((NWORK, ew, DNODE // 2), jnp.int32),
        mesh=plsc.VectorSubcoreMesh(core_axis_name="c", subcore_axis_name="s",
                                    num_cores=NCORE, num_subcores=NSUB),
        scratch_types=[
            pltpu.VMEM((ew,), jnp.int32),
            pltpu.VMEM((GCH, DNODE // 2), jnp.int32),
            pltpu.VMEM((GCH, DNODE // 2), jnp.int32),
            pltpu.SemaphoreType.DMA,
            pltpu.SemaphoreType.DMA,
            pltpu.SemaphoreType.DMA,
            pltpu.SemaphoreType.DMA,
        ],
    )


BE = 4000


def _expand_mats():
    t = np.zeros((3, 3 * MULQ), np.float32)
    u = np.zeros((MULQ, 3 * MULQ), np.float32)
    s = np.zeros((3 * MULQ, MULQ), np.float32)
    for uu in range(MULQ):
        for kk in range(3):
            t[kk, 3 * uu + kk] = 1.0
            u[uu, 3 * uu + kk] = 1.0
            s[3 * uu + kk, uu] = 1.0
    return t, u, s


_TM, _UM, _SM = _expand_mats()


def _ssp(x):
    return jnp.maximum(x, 0.0) + jnp.log(1.0 + jnp.exp(-jnp.abs(x))) - np.float32(np.log(2.0))


def _msg_body(ee_ref, xe_ref, ea_ref, w0_ref, w1_ref, w2_ref, tm_ref, um_ref,
              sm_ref, den_ref, msg_ref):
    f32 = jnp.float32
    ee = ee_ref[...]
    w0 = w0_ref[...] * np.float32(8.0 ** -0.5)
    w1 = w1_ref[...] * np.float32(0.125)
    w2 = w2_ref[...] * np.float32(0.125)
    h = _ssp(jnp.dot(ee, w0, preferred_element_type=f32)) * ACTN
    h = _ssp(jnp.dot(h, w1, preferred_element_type=f32)) * ACTN
    w = jnp.dot(h, w2, preferred_element_type=f32)

    inv_den = 1.0 / den_ref[0, 0]
    w_a = w[:, 0:MULQ]
    w_d = w[:, MULQ:2 * MULQ]
    w_b = w[:, 2 * MULQ:3 * MULQ]
    w_c = w[:, 3 * MULQ:4 * MULQ]

    xe = xe_ref[...].astype(f32)
    xs = xe[:, :MULQ]
    xv = xe[:, MULQ:]
    ea = ea_ref[...]
    f0 = ea[:, 0:1]
    fv = ea[:, 1:4]

    tm = tm_ref[...]
    um = um_ref[...]
    sm = sm_ref[...]
    fve = jnp.dot(fv, tm, preferred_element_type=f32)

    out_a = xs * f0 * w_a
    out_d = jnp.dot(xv * fve, sm, preferred_element_type=f32) \
        * w_d * np.float32(3.0 ** -0.5)
    cb = jnp.dot(jnp.concatenate([xs * w_b, w_c], axis=0), um,
                 preferred_element_type=f32)
    out_b = cb[:BE] * fve
    out_c = xv * f0 * cb[BE:]
    msg_ref[:, 0:MULQ] = out_a * inv_den
    msg_ref[:, MULQ:2 * MULQ] = out_d * inv_den
    msg_ref[:, 2 * MULQ:5 * MULQ] = out_b * inv_den
    msg_ref[:, 5 * MULQ:8 * MULQ] = out_c * inv_den


def _make_msg(ec):
    return pl.pallas_call(
        _msg_body,
        grid=(pl.cdiv(ec, BE),),
        in_specs=[
            pl.BlockSpec((BE, 8), lambda i: (i, 0)),
            pl.BlockSpec((BE, DNODE), lambda i: (i, 0)),
            pl.BlockSpec((BE, 4), lambda i: (i, 0)),
            pl.BlockSpec((8, 64), lambda i: (0, 0)),
            pl.BlockSpec((64, 64), lambda i: (0, 0)),
            pl.BlockSpec((64, 256), lambda i: (0, 0)),
            pl.BlockSpec((3, 192), lambda i: (0, 0)),
            pl.BlockSpec((64, 192), lambda i: (0, 0)),
            pl.BlockSpec((192, 64), lambda i: (0, 0)),
            pl.BlockSpec((1, 1), lambda i: (0, 0), memory_space=pltpu.SMEM),
        ],
        out_specs=pl.BlockSpec((BE, DMSG), lambda i: (i, 0)),
        out_shape=jax.ShapeDtypeStruct((ec, DMSG), jnp.float32),
    )


SCH = 128
RROW = 624
CBLK = 128
NTAIL = NNODE - RROW * NSUB


def _make_scatter(ec, first):
    assert ec % (NSUB * 8) == 0
    et = ec // NSUB
    nfull, tail = divmod(et, SCH)
    npair, odd = divmod(nfull, 2)

    def body(*refs):
        if first:
            (msg, dstidx, out, idx0, idx1, idxt, m0, m1, mt, acc,
             is0, is1, gs0, gs1, ss0, ss1) = refs
            prev = None
        else:
            (msg, dstidx, prev, out, idx0, idx1, idxt, m0, m1, mt, acc,
             is0, is1, gs0, gs1, ss0, ss1) = refs
        cid = lax.axis_index("c")
        sid = lax.axis_index("s")
        ebase = pl.multiple_of(sid * et, 8)
        r0 = pl.multiple_of(sid * RROW, 8)

        for p in range(2):
            col0 = pl.multiple_of((cid * 2 + p) * CBLK, CBLK)
            if first:
                def zfill(r, carry):
                    for j in range(CBLK // 16):
                        m0[r, pl.ds(j * 16, 16)] = jnp.zeros((16,), jnp.float32)
                    return carry

                lax.fori_loop(0, SCH, zfill, 0)
                for q in range(4):
                    pltpu.sync_copy(m0, acc.at[pl.ds(r0 + q * 128, 128)])
                pltpu.sync_copy(m0.at[pl.ds(0, 112)], acc.at[pl.ds(r0 + 512, 112)])

                @pl.when(sid == NSUB - 1)
                def _zero_tail():
                    pltpu.sync_copy(m0.at[pl.ds(0, NTAIL)],
                                    acc.at[pl.ds(NNODE - NTAIL, NTAIL)])
            else:
                pltpu.sync_copy(prev.at[pl.ds(r0, RROW), pl.ds(col0, CBLK)],
                                acc.at[pl.ds(r0, RROW)])

                @pl.when(sid == NSUB - 1)
                def _seed_tail():
                    pltpu.sync_copy(
                        prev.at[pl.ds(NNODE - NTAIL, NTAIL), pl.ds(col0, CBLK)],
                        acc.at[pl.ds(NNODE - NTAIL, NTAIL)])

            plsc.subcore_barrier()

            def chunk_pair(i, carry):
                e0 = pl.multiple_of(ebase + i * (2 * SCH), 8)

                @pl.when(i > 0)
                def _():
                    pltpu.make_async_copy(m0, acc.at[idx0], ss0).wait()

                i0 = pltpu.async_copy(dstidx.at[pl.ds(e0, SCH)], idx0, is0)
                g0 = pltpu.async_copy(msg.at[pl.ds(e0, SCH), pl.ds(col0, CBLK)], m0, gs0)
                i0.wait()
                g0.wait()
                pltpu.async_copy(m0, acc.at[idx0], ss0, add=True)

                @pl.when(i > 0)
                def _():
                    pltpu.make_async_copy(m1, acc.at[idx1], ss1).wait()

                i1 = pltpu.async_copy(dstidx.at[pl.ds(e0 + SCH, SCH)], idx1, is1)
                g1 = pltpu.async_copy(msg.at[pl.ds(e0 + SCH, SCH), pl.ds(col0, CBLK)], m1, gs1)
                i1.wait()
                g1.wait()
                pltpu.async_copy(m1, acc.at[idx1], ss1, add=True)
                return carry

            if npair:
                lax.fori_loop(0, npair, chunk_pair, 0)
                pltpu.make_async_copy(m0, acc.at[idx0], ss0).wait()
                pltpu.make_async_copy(m1, acc.at[idx1], ss1).wait()
            eoff = npair * 2 * SCH
            if odd:
                e1 = pl.multiple_of(ebase + eoff, 8)
                pltpu.sync_copy(dstidx.at[pl.ds(e1, SCH)], idx0)
                pltpu.sync_copy(msg.at[pl.ds(e1, SCH), pl.ds(col0, CBLK)], m0)
                pltpu.sync_copy(m0, acc.at[idx0], add=True)
                eoff += SCH
            if tail:
                e1 = pl.multiple_of(ebase + eoff, 8)
                pltpu.sync_copy(dstidx.at[pl.ds(e1, tail)], idxt)
                pltpu.sync_copy(msg.at[pl.ds(e1, tail), pl.ds(col0, CBLK)], mt)
                pltpu.sync_copy(mt, acc.at[idxt], add=True)
            plsc.subcore_barrier()
            pltpu.sync_copy(acc.at[pl.ds(r0, RROW)],
                            out.at[pl.ds(r0, RROW), pl.ds(col0, CBLK)])

            @pl.when(sid == NSUB - 1)
            def _wb_tail():
                pltpu.sync_copy(acc.at[pl.ds(NNODE - NTAIL, NTAIL)],
                                out.at[pl.ds(NNODE - NTAIL, NTAIL), pl.ds(col0, CBLK)])

            plsc.subcore_barrier()

    tail_n = max(tail, 8)
    scratch = [
        pltpu.VMEM((SCH,), jnp.int32),
        pltpu.VMEM((SCH,), jnp.int32),
        pltpu.VMEM((tail_n,), jnp.int32),
        pltpu.VMEM((SCH, CBLK), jnp.float32),
        pltpu.VMEM((SCH, CBLK), jnp.float32),
        pltpu.VMEM((tail_n, CBLK), jnp.float32),
    ]
    scratch.append(pltpu.VMEM_SHARED((NNODE, CBLK), jnp.float32))
    scratch += [pltpu.SemaphoreType.DMA] * 6

    return pl.kernel(
        body,
        out_type=jax.ShapeDtypeStruct((NNODE, DMSG), jnp.float32),
        mesh=plsc.VectorSubcoreMesh(core_axis_name="c", subcore_axis_name="s",
                                    num_cores=NCORE, num_subcores=NSUB),
        scratch_types=scratch,
    )


_GATHERS = {ec: _make_gather(ec) for ec in set(CHUNKS)}
_MSGS = {ec: _make_msg(ec) for ec in set(CHUNKS)}
_SCATTERS = [_make_scatter(ec, i == 0) for i, ec in enumerate(CHUNKS)]


def kernel(node_feature, edge_attr, edge_embedding, edge_index, W0, W1, W2, denominator):
    edge_src = edge_index[1]
    edge_dst = edge_index[0]
    den = denominator.reshape(1, 1)
    tm, um, sm = jnp.asarray(_TM), jnp.asarray(_UM), jnp.asarray(_SM)
    nf16 = node_feature.astype(jnp.bfloat16)
    nfi = lax.bitcast_convert_type(nf16.reshape(NNODE, DNODE // 2, 2), jnp.int32)

    out = None
    off = 0
    for i, ec in enumerate(CHUNKS):
        src_c = lax.dynamic_slice_in_dim(edge_src, off, ec)
        dst_c = lax.dynamic_slice_in_dim(edge_dst, off, ec)
        ea_c = lax.dynamic_slice_in_dim(edge_attr, off, ec)
        ee_c = lax.dynamic_slice_in_dim(edge_embedding, off, ec)
        xe_i = _GATHERS[ec](nfi, src_c).reshape(ec, DNODE // 2)
        xe = lax.bitcast_convert_type(xe_i, jnp.bfloat16).reshape(ec, DNODE)
        msg = _MSGS[ec](ee_c, xe, ea_c, W0, W1, W2, tm, um, sm, den)
        if i == 0:
            out = _SCATTERS[i](msg, dst_c)
        else:
            out = _SCATTERS[i](msg, dst_c, out)
        off += ec
    return out


# speedup vs baseline: 2.2271x; 2.2271x over previous
import numpy as np
import jax
import jax.numpy as jnp
from jax import lax
from jax.experimental import pallas as pl
from jax.experimental.pallas import tpu as pltpu
from jax.experimental.pallas import tpu_sc as plsc

NNODE = 10000
NEDGE = 160000
MULQ = 64
DNODE = 256
DMSG = 512
ACTN = 1.679177

NCORE = 2
NSUB = 16
NWORK = NCORE * NSUB

CHUNKS = (80128, 79872)


GCH = 128


def _make_gather(ec):
    assert ec % (NWORK * 8) == 0
    ew = ec // NWORK
    nfull, tail = divmod(ew, GCH)
    npair, odd = divmod(nfull, 2)

    def body(nodes, srcidx, xe, idx_v, rows0, rows1, gs0, gs1, ws0, ws1):
        cid = lax.axis_index("c")
        sid = lax.axis_index("s")
        wid = sid * NCORE + cid
        base = pl.multiple_of(wid * ew, 8)
        pltpu.sync_copy(srcidx.at[pl.ds(base, ew)], idx_v)

        def loop(i, carry):
            off = pl.multiple_of(i * (2 * GCH), 2 * GCH)

            @pl.when(i > 0)
            def _():
                pltpu.make_async_copy(rows0, xe.at[pl.ds(base, GCH)], ws0).wait()

            g0 = pltpu.async_copy(nodes.at[idx_v.at[pl.ds(off, GCH)]], rows0, gs0)

            @pl.when(i > 0)
            def _():
                pltpu.make_async_copy(rows1, xe.at[pl.ds(base, GCH)], ws1).wait()

            g1 = pltpu.async_copy(nodes.at[idx_v.at[pl.ds(off + GCH, GCH)]], rows1, gs1)
            g0.wait()
            pltpu.async_copy(rows0, xe.at[pl.ds(base + off, GCH)], ws0)
            g1.wait()
            pltpu.async_copy(rows1, xe.at[pl.ds(base + off + GCH, GCH)], ws1)
            return carry

        if npair:
            lax.fori_loop(0, npair, loop, 0)
            pltpu.make_async_copy(rows0, xe.at[pl.ds(base, GCH)], ws0).wait()
            pltpu.make_async_copy(rows1, xe.at[pl.ds(base, GCH)], ws1).wait()
        off = npair * 2 * GCH
        if odd:
            g = pltpu.async_copy(nodes.at[idx_v.at[pl.ds(off, GCH)]], rows0, gs0)
            g.wait()
            pltpu.sync_copy(rows0, xe.at[pl.ds(base + off, GCH)])
            off += GCH
        if tail:
            g = pltpu.async_copy(nodes.at[idx_v.at[pl.ds(off, tail)]],
                                 rows1.at[pl.ds(0, tail)], gs1)
            g.wait()
            pltpu.sync_copy(rows1.at[pl.ds(0, tail)], xe.at[pl.ds(base + off, tail)])

    return pl.kernel(
        body,
        out_type=jax.ShapeDtypeStruct((ec, DNODE), jnp.float32),
        mesh=plsc.VectorSubcoreMesh(core_axis_name="c", subcore_axis_name="s",
                                    num_cores=NCORE, num_subcores=NSUB),
        scratch_types=[
            pltpu.VMEM((ew,), jnp.int32),
            pltpu.VMEM((GCH, DNODE), jnp.float32),
            pltpu.VMEM((GCH, DNODE), jnp.float32),
            pltpu.SemaphoreType.DMA,
            pltpu.SemaphoreType.DMA,
            pltpu.SemaphoreType.DMA,
            pltpu.SemaphoreType.DMA,
        ],
    )


BE = 4000


def _expand_mats():
    t = np.zeros((3, 3 * MULQ), np.float32)
    u = np.zeros((MULQ, 3 * MULQ), np.float32)
    s = np.zeros((3 * MULQ, MULQ), np.float32)
    for uu in range(MULQ):
        for kk in range(3):
            t[kk, 3 * uu + kk] = 1.0
            u[uu, 3 * uu + kk] = 1.0
            s[3 * uu + kk, uu] = 1.0
    return t, u, s


_TM, _UM, _SM = _expand_mats()


def _ssp(x):
    return jnp.maximum(x, 0.0) + jnp.log(1.0 + jnp.exp(-jnp.abs(x))) - np.float32(np.log(2.0))


def _msg_body(ee_ref, xe_ref, ea_ref, w0_ref, w1_ref, w2_ref, tm_ref, um_ref,
              sm_ref, den_ref, msg_ref):
    f32 = jnp.float32
    ee = ee_ref[...]
    w0 = w0_ref[...] * np.float32(8.0 ** -0.5)
    w1 = w1_ref[...] * np.float32(0.125)
    w2 = w2_ref[...] * np.float32(0.125)
    h = _ssp(jnp.dot(ee, w0, preferred_element_type=f32)) * ACTN
    h = _ssp(jnp.dot(h, w1, preferred_element_type=f32)) * ACTN
    w = jnp.dot(h, w2, preferred_element_type=f32)

    inv_den = 1.0 / den_ref[0, 0]
    w_a = w[:, 0:MULQ]
    w_d = w[:, MULQ:2 * MULQ]
    w_b = w[:, 2 * MULQ:3 * MULQ]
    w_c = w[:, 3 * MULQ:4 * MULQ]

    xe = xe_ref[...]
    xs = xe[:, :MULQ]
    xv = xe[:, MULQ:]
    ea = ea_ref[...]
    f0 = ea[:, 0:1]
    fv = ea[:, 1:4]

    tm = tm_ref[...]
    um = um_ref[...]
    sm = sm_ref[...]
    fve = jnp.dot(fv, tm, preferred_element_type=f32)

    out_a = xs * f0 * w_a
    out_d = jnp.dot(xv * fve, sm, preferred_element_type=f32) \
        * w_d * np.float32(3.0 ** -0.5)
    cb = jnp.dot(jnp.concatenate([xs * w_b, w_c], axis=0), um,
                 preferred_element_type=f32)
    out_b = cb[:BE] * fve
    out_c = xv * f0 * cb[BE:]
    msg_ref[0, :, 0:MULQ] = out_a * inv_den
    msg_ref[0, :, MULQ:2 * MULQ] = out_d * inv_den
    msg_ref[1, :, :] = out_b[:, 0:2 * MULQ] * inv_den
    msg_ref[2, :, 0:MULQ] = out_b[:, 2 * MULQ:] * inv_den
    msg_ref[2, :, MULQ:2 * MULQ] = out_c[:, 0:MULQ] * inv_den
    msg_ref[3, :, :] = out_c[:, MULQ:] * inv_den


def _make_msg(ec):
    return pl.pallas_call(
        _msg_body,
        grid=(pl.cdiv(ec, BE),),
        in_specs=[
            pl.BlockSpec((BE, 8), lambda i: (i, 0)),
            pl.BlockSpec((BE, DNODE), lambda i: (i, 0)),
            pl.BlockSpec((BE, 4), lambda i: (i, 0)),
            pl.BlockSpec((8, 64), lambda i: (0, 0)),
            pl.BlockSpec((64, 64), lambda i: (0, 0)),
            pl.BlockSpec((64, 256), lambda i: (0, 0)),
            pl.BlockSpec((3, 192), lambda i: (0, 0)),
            pl.BlockSpec((64, 192), lambda i: (0, 0)),
            pl.BlockSpec((192, 64), lambda i: (0, 0)),
            pl.BlockSpec((1, 1), lambda i: (0, 0), memory_space=pltpu.SMEM),
        ],
        out_specs=pl.BlockSpec((4, BE, CBLK), lambda i: (0, i, 0)),
        out_shape=jax.ShapeDtypeStruct((4, ec, CBLK), jnp.float32),
    )


SCH = 128
RROW = 624
CBLK = 128
NTAIL = NNODE - RROW * NSUB


def _make_scatter(ec, first):
    assert ec % (NSUB * 8) == 0
    et = ec // NSUB
    nfull, tail = divmod(et, SCH)
    npair, odd = divmod(nfull, 2)

    def body(*refs):
        if first:
            (msg, dstidx, out, idx0, idx1, idxt, m0, m1, mt, acc,
             is0, is1, gs0, gs1, ss0, ss1) = refs
            prev = None
        else:
            (msg, dstidx, prev, out, idx0, idx1, idxt, m0, m1, mt, acc,
             is0, is1, gs0, gs1, ss0, ss1) = refs
        cid = lax.axis_index("c")
        sid = lax.axis_index("s")
        ebase = pl.multiple_of(sid * et, 8)
        r0 = pl.multiple_of(sid * RROW, 8)

        for p in range(2):
            blk = cid * 2 + p
            col0 = pl.multiple_of(blk * CBLK, CBLK)
            if first:
                def zfill(r, carry):
                    for j in range(CBLK // 16):
                        m0[r, pl.ds(j * 16, 16)] = jnp.zeros((16,), jnp.float32)
                    return carry

                lax.fori_loop(0, SCH, zfill, 0)
                for q in range(4):
                    pltpu.sync_copy(m0, acc.at[pl.ds(r0 + q * 128, 128)])
                pltpu.sync_copy(m0.at[pl.ds(0, 112)], acc.at[pl.ds(r0 + 512, 112)])

                @pl.when(sid == NSUB - 1)
                def _zero_tail():
                    pltpu.sync_copy(m0.at[pl.ds(0, NTAIL)],
                                    acc.at[pl.ds(NNODE - NTAIL, NTAIL)])
            else:
                pltpu.sync_copy(prev.at[pl.ds(r0, RROW), pl.ds(col0, CBLK)],
                                acc.at[pl.ds(r0, RROW)])

                @pl.when(sid == NSUB - 1)
                def _seed_tail():
                    pltpu.sync_copy(
                        prev.at[pl.ds(NNODE - NTAIL, NTAIL), pl.ds(col0, CBLK)],
                        acc.at[pl.ds(NNODE - NTAIL, NTAIL)])

            plsc.subcore_barrier()

            def chunk_pair(i, carry):
                e0 = pl.multiple_of(ebase + i * (2 * SCH), 8)

                @pl.when(i > 0)
                def _():
                    pltpu.make_async_copy(m0, acc.at[idx0], ss0).wait()

                i0 = pltpu.async_copy(dstidx.at[pl.ds(e0, SCH)], idx0, is0)
                g0 = pltpu.async_copy(msg.at[blk, pl.ds(e0, SCH)], m0, gs0)
                i0.wait()
                g0.wait()
                pltpu.async_copy(m0, acc.at[idx0], ss0, add=True)

                @pl.when(i > 0)
                def _():
                    pltpu.make_async_copy(m1, acc.at[idx1], ss1).wait()

                i1 = pltpu.async_copy(dstidx.at[pl.ds(e0 + SCH, SCH)], idx1, is1)
                g1 = pltpu.async_copy(msg.at[blk, pl.ds(e0 + SCH, SCH)], m1, gs1)
                i1.wait()
                g1.wait()
                pltpu.async_copy(m1, acc.at[idx1], ss1, add=True)
                return carry

            if npair:
                lax.fori_loop(0, npair, chunk_pair, 0)
                pltpu.make_async_copy(m0, acc.at[idx0], ss0).wait()
                pltpu.make_async_copy(m1, acc.at[idx1], ss1).wait()
            eoff = npair * 2 * SCH
            if odd:
                e1 = pl.multiple_of(ebase + eoff, 8)
                pltpu.sync_copy(dstidx.at[pl.ds(e1, SCH)], idx0)
                pltpu.sync_copy(msg.at[blk, pl.ds(e1, SCH)], m0)
                pltpu.sync_copy(m0, acc.at[idx0], add=True)
                eoff += SCH
            if tail:
                e1 = pl.multiple_of(ebase + eoff, 8)
                pltpu.sync_copy(dstidx.at[pl.ds(e1, tail)], idxt)
                pltpu.sync_copy(msg.at[blk, pl.ds(e1, tail)], mt)
                pltpu.sync_copy(mt, acc.at[idxt], add=True)
            plsc.subcore_barrier()
            pltpu.sync_copy(acc.at[pl.ds(r0, RROW)],
                            out.at[pl.ds(r0, RROW), pl.ds(col0, CBLK)])

            @pl.when(sid == NSUB - 1)
            def _wb_tail():
                pltpu.sync_copy(acc.at[pl.ds(NNODE - NTAIL, NTAIL)],
                                out.at[pl.ds(NNODE - NTAIL, NTAIL), pl.ds(col0, CBLK)])

            plsc.subcore_barrier()

    tail_n = max(tail, 8)
    scratch = [
        pltpu.VMEM((SCH,), jnp.int32),
        pltpu.VMEM((SCH,), jnp.int32),
        pltpu.VMEM((tail_n,), jnp.int32),
        pltpu.VMEM((SCH, CBLK), jnp.float32),
        pltpu.VMEM((SCH, CBLK), jnp.float32),
        pltpu.VMEM((tail_n, CBLK), jnp.float32),
    ]
    scratch.append(pltpu.VMEM_SHARED((NNODE, CBLK), jnp.float32))
    scratch += [pltpu.SemaphoreType.DMA] * 6

    return pl.kernel(
        body,
        out_type=jax.ShapeDtypeStruct((NNODE, DMSG), jnp.float32),
        mesh=plsc.VectorSubcoreMesh(core_axis_name="c", subcore_axis_name="s",
                                    num_cores=NCORE, num_subcores=NSUB),
        scratch_types=scratch,
    )


_GATHERS = {ec: _make_gather(ec) for ec in set(CHUNKS)}
_MSGS = {ec: _make_msg(ec) for ec in set(CHUNKS)}
_SCATTERS = [_make_scatter(ec, i == 0) for i, ec in enumerate(CHUNKS)]


def kernel(node_feature, edge_attr, edge_embedding, edge_index, W0, W1, W2, denominator):
    edge_src = edge_index[1]
    edge_dst = edge_index[0]
    den = denominator.reshape(1, 1)
    tm, um, sm = jnp.asarray(_TM), jnp.asarray(_UM), jnp.asarray(_SM)

    out = None
    off = 0
    for i, ec in enumerate(CHUNKS):
        src_c = lax.dynamic_slice_in_dim(edge_src, off, ec)
        dst_c = lax.dynamic_slice_in_dim(edge_dst, off, ec)
        ea_c = lax.dynamic_slice_in_dim(edge_attr, off, ec)
        ee_c = lax.dynamic_slice_in_dim(edge_embedding, off, ec)
        xe = _GATHERS[ec](node_feature, src_c)
        msg = _MSGS[ec](ee_c, xe, ea_c, W0, W1, W2, tm, um, sm, den)
        if i == 0:
            out = _SCATTERS[i](msg, dst_c)
        else:
            out = _SCATTERS[i](msg, dst_c, out)
        off += ec
    return out
